# baseline (speedup 1.0000x reference)
import functools

import jax
from jax import lax
from jax.experimental import pallas as pl
from jax.experimental.pallas import tpu as pltpu
from jax.experimental.pallas import tpu_sc as plsc

_LANES = 128


def _feats_body(f_in, f_out):
    f_out[...] = f_in[...]


def _copy_feats(feats):
    n, d = feats.shape
    grid = 5
    return pl.pallas_call(
        _feats_body,
        grid=(grid,),
        in_specs=[pl.BlockSpec((n // grid, d), lambda i: (i, 0))],
        out_specs=pl.BlockSpec((n // grid, d), lambda i: (i, 0)),
        out_shape=jax.ShapeDtypeStruct(feats.shape, feats.dtype),
        compiler_params=pltpu.CompilerParams(
            dimension_semantics=("arbitrary",),
        ),
    )(feats)


def _make_sc_copy(ei_shape, ea_shape, ei_dtype, ea_dtype, nc, ns):
    nw = nc * ns

    def split(rows):
        per = (rows // nw) // 8 * 8
        rem = rows - per * nw
        chunk = per
        for c in range(min(per, 256), 7, -8):
            if per % c == 0:
                chunk = c
                break
        return per, rem, chunk

    ei_per, ei_rem, ei_chunk = split(ei_shape[0])
    ea_per, ea_rem, ea_chunk = split(ea_shape[0])
    mesh = plsc.VectorSubcoreMesh(core_axis_name="c", subcore_axis_name="s")

    @functools.partial(
        pl.kernel,
        mesh=mesh,
        out_type=[
            jax.ShapeDtypeStruct(ei_shape, ei_dtype),
            jax.ShapeDtypeStruct(ea_shape, ea_dtype),
        ],
        scratch_types=[
            pltpu.VMEM((ei_chunk, _LANES), ei_dtype),
            pltpu.VMEM((ei_chunk, _LANES), ei_dtype),
            pltpu.VMEM((ea_chunk, _LANES), ea_dtype),
            pltpu.VMEM((ea_chunk, _LANES), ea_dtype),
            pltpu.SemaphoreType.DMA((2, 2)),
            pltpu.SemaphoreType.DMA((2, 2)),
        ],
    )
    def sc_copy(ei_hbm, ea_hbm, ei_out, ea_out, ei_v0, ei_v1, ea_v0, ea_v1, in_sem, out_sem):
        wid = lax.axis_index("s") * nc + lax.axis_index("c")

        def copy_array(src, dst, bufs, per, rem, chunk, arr):
            nchunks = per // chunk
            base = wid * per

            def start_in(j, b, rows):
                pltpu.async_copy(src.at[pl.ds(base + j * chunk, rows)],
                                 bufs[b].at[pl.ds(0, rows)], in_sem.at[b, arr])

            def wait_in(b, rows):
                pltpu.make_async_copy(src.at[pl.ds(base, rows)],
                                      bufs[b].at[pl.ds(0, rows)], in_sem.at[b, arr]).wait()

            def start_out(j, b, rows):
                pltpu.async_copy(bufs[b].at[pl.ds(0, rows)],
                                 dst.at[pl.ds(base + j * chunk, rows)], out_sem.at[b, arr])

            def wait_out(b, rows):
                pltpu.make_async_copy(bufs[b].at[pl.ds(0, rows)],
                                      dst.at[pl.ds(base, rows)], out_sem.at[b, arr]).wait()

            start_in(0, 0, chunk)
            if nchunks > 1:
                start_in(1, 1, chunk)
            for j in range(nchunks):
                b = j % 2
                wait_in(b, chunk)
                start_out(j, b, chunk)
                if j + 2 < nchunks:
                    wait_out(b, chunk)
                    start_in(j + 2, b, chunk)
            wait_out((nchunks - 1) % 2, chunk)
            if nchunks > 1:
                wait_out(nchunks % 2, chunk)

            if rem:
                tail = src.shape[0] - rem

                @pl.when(wid < rem // 8)
                def _tail():
                    o = tail + wid * 8
                    pltpu.async_copy(src.at[pl.ds(o, 8)],
                                     bufs[0].at[pl.ds(0, 8)], in_sem.at[0, arr])
                    pltpu.make_async_copy(src.at[pl.ds(tail, 8)],
                                          bufs[0].at[pl.ds(0, 8)], in_sem.at[0, arr]).wait()
                    pltpu.async_copy(bufs[0].at[pl.ds(0, 8)],
                                     dst.at[pl.ds(o, 8)], out_sem.at[0, arr])
                    pltpu.make_async_copy(bufs[0].at[pl.ds(0, 8)],
                                          dst.at[pl.ds(tail, 8)], out_sem.at[0, arr]).wait()

        copy_array(ei_hbm, ei_out, (ei_v0, ei_v1), ei_per, ei_rem, ei_chunk, 0)
        copy_array(ea_hbm, ea_out, (ea_v0, ea_v1), ea_per, ea_rem, ea_chunk, 1)

    return sc_copy


def kernel(feats, edge_index, edge_attr):
    e, ik = edge_index.shape
    _, ak = edge_attr.shape

    ei2 = edge_index.reshape((e * ik) // _LANES, _LANES)
    ea2 = edge_attr.reshape((e * ak) // _LANES, _LANES)

    info = plsc.get_sparse_core_info()
    sc_copy = _make_sc_copy(ei2.shape, ea2.shape, ei2.dtype, ea2.dtype,
                            info.num_cores, info.num_subcores)
    ei_o, ea_o = sc_copy(ei2, ea2)
    f_o = _copy_feats(feats)
    return (f_o, ei_o.reshape(e, ik), ea_o.reshape(e, ak))
